# Initial kernel scaffold; baseline (speedup 1.0000x reference)
import functools

import jax
import jax.numpy as jnp
from jax import lax
from jax.experimental import pallas as pl
from jax.experimental.pallas import tpu as pltpu
from jax.experimental.pallas import tpu_sc as plsc

N = 10000
DIM = 128
ENC = 256
E = 320000

NC = 2
NS = 16
L = 16
NW = NC * NS
EPW = E // NW
EK = 80
NCHUNK = EPW // EK
DK = 2000
NDCHUNK = EPW // DK
RPT = N // NS

_EPS = 1e-5


def _bn(x, g, b):
    mu = jnp.mean(x, axis=0, keepdims=True)
    var = jnp.mean((x - mu) * (x - mu), axis=0, keepdims=True)
    return (x - mu) * lax.rsqrt(var + _EPS) * g + b


def _encoder_body(x_ref, w1_ref, b1_ref, g1_ref, be1_ref, w2_ref, b2_ref,
                  g2_ref, be2_ref, h_ref):
    h1 = jnp.dot(x_ref[...], w1_ref[...], preferred_element_type=jnp.float32)
    h1 = h1 + b1_ref[...]
    h1 = jnp.maximum(_bn(h1, g1_ref[...], be1_ref[...]), 0.0)
    h2 = jnp.dot(h1, w2_ref[...], preferred_element_type=jnp.float32)
    h2 = h2 + b2_ref[...]
    h_ref[...] = jnp.maximum(_bn(h2, g2_ref[...], be2_ref[...]), 0.0)


def _encoder(x, w1t, b1, g1, be1, w2t, b2, g2, be2):
    return pl.pallas_call(
        _encoder_body,
        out_shape=jax.ShapeDtypeStruct((N, DIM), jnp.float32),
    )(x, w1t, b1, g1, be1, w2t, b2, g2, be2)


_RB = 2500


def _tables_body(h_ref, wa_ref, wb_ref, b1_ref, w_ref, a_ref, b_ref, c_ref):
    h = h_ref[...]
    a_ref[...] = jnp.dot(h, wa_ref[...], preferred_element_type=jnp.float32)
    b_ref[...] = jnp.dot(h, wb_ref[...],
                         preferred_element_type=jnp.float32) + b1_ref[...]
    c_ref[...] = jnp.dot(h, w_ref[...], preferred_element_type=jnp.float32)


def _tables(h, wat, wbt, b1, wt):
    row_spec = pl.BlockSpec((_RB, DIM), lambda i: (i, 0))
    full = pl.BlockSpec((DIM, DIM), lambda i: (0, 0))
    bias = pl.BlockSpec((1, DIM), lambda i: (0, 0))
    out = jax.ShapeDtypeStruct((N, DIM), jnp.float32)
    return pl.pallas_call(
        _tables_body,
        grid=(N // _RB,),
        in_specs=[row_spec, full, full, bias, full],
        out_specs=[row_spec, row_spec, row_spec],
        out_shape=[out, out, out],
    )(h, wat, wbt, b1, wt)


def _finalize_body(h_ref, p_ref, deg_ref, g_ref, b_ref, o_ref):
    d = jnp.sum(deg_ref[...], axis=1, keepdims=True)
    inv = 1.0 / jnp.maximum(d, 1.0)
    t = h_ref[...] + (p_ref[0] + p_ref[1]) * inv
    mu = jnp.mean(t, axis=1, keepdims=True)
    var = jnp.mean((t - mu) * (t - mu), axis=1, keepdims=True)
    ln = (t - mu) * lax.rsqrt(var + _EPS) * g_ref[...] + b_ref[...]
    o_ref[...] = jnp.maximum(ln, 0.0)


def _finalize(h, part, degt, ln_g, ln_b):
    row_spec = pl.BlockSpec((_RB, DIM), lambda i: (i, 0))
    return pl.pallas_call(
        _finalize_body,
        grid=(N // _RB,),
        in_specs=[
            row_spec,
            pl.BlockSpec((NC, _RB, DIM), lambda i: (0, i, 0)),
            pl.BlockSpec((_RB, NW), lambda i: (i, 0)),
            pl.BlockSpec((1, DIM), lambda i: (0, 0)),
            pl.BlockSpec((1, DIM), lambda i: (0, 0)),
        ],
        out_specs=row_spec,
        out_shape=jax.ShapeDtypeStruct((N, DIM), jnp.float32),
    )(h, part, degt, ln_g, ln_b)


def _head_body(h_ref, w1_ref, b1_ref, w2_ref, b2_ref, y_ref):
    t = jnp.dot(h_ref[...], w1_ref[...], preferred_element_type=jnp.float32)
    t = jnp.maximum(t + b1_ref[...], 0.0)
    y = jnp.dot(t, w2_ref[...], preferred_element_type=jnp.float32)
    y_ref[...] = y + b2_ref[...]


def _head(h, w1t, b1, w2t, b2):
    return pl.pallas_call(
        _head_body,
        grid=(N // _RB,),
        in_specs=[
            pl.BlockSpec((_RB, DIM), lambda i: (i, 0)),
            pl.BlockSpec((DIM, DIM // 2), lambda i: (0, 0)),
            pl.BlockSpec((1, DIM // 2), lambda i: (0, 0)),
            pl.BlockSpec((DIM // 2, 1), lambda i: (0, 0)),
            pl.BlockSpec((1, 1), lambda i: (0, 0)),
        ],
        out_specs=pl.BlockSpec((_RB, 1), lambda i: (i, 0)),
        out_shape=jax.ShapeDtypeStruct((N, 1), jnp.float32),
    )(h, w1t, b1, w2t, b2)


def _sc_mesh():
    return plsc.VectorSubcoreMesh(core_axis_name="c", subcore_axis_name="s")


def _deg_body(row_hbm, deg_hbm, hist, idxv):
    cid = lax.axis_index("c")
    sid = lax.axis_index("s")
    wid = sid * NC + cid
    zeros = jnp.zeros((L,), jnp.float32)
    ones = jnp.ones((L,), jnp.float32)

    def zero_body(i, _):
        hist[pl.ds(i * L, L)] = zeros
        return 0

    lax.fori_loop(0, N // L, zero_body, 0)

    def chunk_body(c, _):
        pltpu.sync_copy(row_hbm.at[pl.ds(wid * EPW + c * DK, DK)], idxv)

        def edge_body(i, _):
            iv = idxv[pl.ds(i * L, L)]
            plsc.addupdate_scatter(hist, [iv], ones)
            return 0

        lax.fori_loop(0, DK // L, edge_body, 0)
        return 0

    lax.fori_loop(0, NDCHUNK, chunk_body, 0)
    pltpu.sync_copy(hist, deg_hbm.at[wid])


def _degrees(row):
    fn = functools.partial(
        pl.kernel,
        out_type=jax.ShapeDtypeStruct((NW, N), jnp.float32),
        mesh=_sc_mesh(),
        scratch_types=[
            pltpu.VMEM((N,), jnp.float32),
            pltpu.VMEM((DK,), jnp.int32),
        ],
    )(_deg_body)
    return fn(row)


def _edge_body(a_hbm, b_hbm, c_hbm, row_hbm, col_hbm, w2_hbm, b2_hbm,
               zero_hbm, out_hbm, row_v, col_v, av, bv, cv, w2v, b2v,
               acc, sem):
    cid = lax.axis_index("c")
    sid = lax.axis_index("s")
    wid = sid * NC + cid

    pltpu.sync_copy(zero_hbm.at[pl.ds(sid * RPT, RPT), :],
                    acc.at[pl.ds(sid * RPT, RPT), :])
    pltpu.sync_copy(w2_hbm, w2v)
    pltpu.sync_copy(b2_hbm, b2v)
    plsc.subcore_barrier()

    base = wid * EPW

    def chunk_body(gi, _):
        off = base + gi * EK
        pltpu.sync_copy(row_hbm.at[pl.ds(off, EK)], row_v)
        pltpu.sync_copy(col_hbm.at[pl.ds(off, EK)], col_v)
        ga = pltpu.async_copy(a_hbm.at[row_v], av, sem)
        gb = pltpu.async_copy(b_hbm.at[col_v], bv, sem)
        gc = pltpu.async_copy(c_hbm.at[col_v], cv, sem)
        ga.wait()
        gb.wait()
        gc.wait()
        b2l = b2v[...]

        def edge_body(e, _):
            acc16 = jnp.zeros((L,), jnp.float32)
            for j in range(DIM // L):
                sl = pl.ds(j * L, L)
                acc16 = acc16 + (jnp.maximum(av[e, sl] + bv[e, sl], 0.0)
                                 * w2v[sl])
            z = jnp.sum(acc16)
            gate = 1.0 / (1.0 + jnp.exp(-(jnp.full((L,), z) + b2l)))
            for j in range(DIM // L):
                sl = pl.ds(j * L, L)
                cv[e, sl] = cv[e, sl] * gate
            return 0

        lax.fori_loop(0, EK, edge_body, 0)
        pltpu.sync_copy(cv, acc.at[row_v], add=True)
        return 0

    lax.fori_loop(0, NCHUNK, chunk_body, 0)
    plsc.subcore_barrier()
    pltpu.sync_copy(acc.at[pl.ds(sid * RPT, RPT), :],
                    out_hbm.at[cid, pl.ds(sid * RPT, RPT), :])


def _edge_stage(a, b, c, row, col, w2, b2v, zeros):
    fn = functools.partial(
        pl.kernel,
        out_type=jax.ShapeDtypeStruct((NC, N, DIM), jnp.float32),
        mesh=_sc_mesh(),
        scratch_types=[
            pltpu.VMEM((EK,), jnp.int32),
            pltpu.VMEM((EK,), jnp.int32),
            pltpu.VMEM((EK, DIM), jnp.float32),
            pltpu.VMEM((EK, DIM), jnp.float32),
            pltpu.VMEM((EK, DIM), jnp.float32),
            pltpu.VMEM((DIM,), jnp.float32),
            pltpu.VMEM((L,), jnp.float32),
            pltpu.VMEM_SHARED((N, DIM), jnp.float32),
            pltpu.SemaphoreType.DMA,
        ],
    )(_edge_body)
    return fn(a, b, c, row, col, w2, b2v, zeros)


def kernel(x, edge_index, params):
    p = params
    row = edge_index[0]
    col = edge_index[1]

    h = _encoder(
        x,
        p['enc_W1'].T, p['enc_b1'][None, :], p['bn1_g'][None, :],
        p['bn1_b'][None, :],
        p['enc_W2'].T, p['enc_b2'][None, :], p['bn2_g'][None, :],
        p['bn2_b'][None, :],
    )

    deg = _degrees(row)
    degt = deg.T
    zeros = jnp.zeros((N, DIM), jnp.float32)

    for lp in p['layers']:
        wat = lp['g_W1'][:, :DIM].T
        wbt = lp['g_W1'][:, DIM:].T
        wt = lp['W'].T
        a, b, c = _tables(h, wat, wbt, lp['g_b1'][None, :], wt)
        b2v = jnp.full((L,), lp['g_b2'][0], jnp.float32)
        part = _edge_stage(a, b, c, row, col, lp['g_W2'][0], b2v, zeros)
        h = _finalize(h, part, degt, lp['ln_g'][None, :], lp['ln_b'][None, :])

    return _head(h, p['head_W1'].T, p['head_b1'][None, :],
                 p['head_W2'].T, p['head_b2'][None, :])


# speedup vs baseline: 2.7505x; 2.7505x over previous
import functools

import jax
import jax.numpy as jnp
from jax import lax
from jax.experimental import pallas as pl
from jax.experimental.pallas import tpu as pltpu
from jax.experimental.pallas import tpu_sc as plsc

N = 10000
DIM = 128
ENC = 256
E = 320000

NC = 2
NS = 16
L = 16
NW = NC * NS
EPW = E // NW
EK = 80
NCHUNK = EPW // EK
DK = 2000
NDCHUNK = EPW // DK
NP = 10240
RPT = NP // NS

_EPS = 1e-5


def _bn(x, g, b):
    mu = jnp.mean(x, axis=0, keepdims=True)
    var = jnp.mean((x - mu) * (x - mu), axis=0, keepdims=True)
    return (x - mu) * lax.rsqrt(var + _EPS) * g + b


def _encoder_body(x_ref, w1_ref, b1_ref, g1_ref, be1_ref, w2_ref, b2_ref,
                  g2_ref, be2_ref, h_ref):
    h1 = jnp.dot(x_ref[...], w1_ref[...], preferred_element_type=jnp.float32)
    h1 = h1 + b1_ref[...]
    h1 = jnp.maximum(_bn(h1, g1_ref[...], be1_ref[...]), 0.0)
    h2 = jnp.dot(h1, w2_ref[...], preferred_element_type=jnp.float32)
    h2 = h2 + b2_ref[...]
    h_ref[...] = jnp.maximum(_bn(h2, g2_ref[...], be2_ref[...]), 0.0)


def _encoder(x, w1t, b1, g1, be1, w2t, b2, g2, be2):
    return pl.pallas_call(
        _encoder_body,
        out_shape=jax.ShapeDtypeStruct((N, DIM), jnp.float32),
    )(x, w1t, b1, g1, be1, w2t, b2, g2, be2)


_RB = 2000


def _tables_body(h_ref, wa_ref, wb_ref, b1_ref, w_ref, a_ref, b_ref, c_ref):
    h = h_ref[...]
    a_ref[...] = jnp.dot(h, wa_ref[...], preferred_element_type=jnp.float32)
    b_ref[...] = jnp.dot(h, wb_ref[...],
                         preferred_element_type=jnp.float32) + b1_ref[...]
    c_ref[...] = jnp.dot(h, w_ref[...], preferred_element_type=jnp.float32)


def _tables(h, wat, wbt, b1, wt):
    row_spec = pl.BlockSpec((_RB, DIM), lambda i: (i, 0))
    full = pl.BlockSpec((DIM, DIM), lambda i: (0, 0))
    bias = pl.BlockSpec((1, DIM), lambda i: (0, 0))
    out = jax.ShapeDtypeStruct((N, DIM), jnp.float32)
    return pl.pallas_call(
        _tables_body,
        grid=(N // _RB,),
        in_specs=[row_spec, full, full, bias, full],
        out_specs=[row_spec, row_spec, row_spec],
        out_shape=[out, out, out],
    )(h, wat, wbt, b1, wt)


def _finalize_body(h_ref, p_ref, deg_ref, g_ref, b_ref, o_ref):
    d = jnp.sum(deg_ref[...], axis=1, keepdims=True)
    inv = 1.0 / jnp.maximum(d, 1.0)
    t = h_ref[...] + (p_ref[0] + p_ref[1]) * inv
    mu = jnp.mean(t, axis=1, keepdims=True)
    var = jnp.mean((t - mu) * (t - mu), axis=1, keepdims=True)
    ln = (t - mu) * lax.rsqrt(var + _EPS) * g_ref[...] + b_ref[...]
    o_ref[...] = jnp.maximum(ln, 0.0)


def _finalize(h, part, degt, ln_g, ln_b):
    row_spec = pl.BlockSpec((_RB, DIM), lambda i: (i, 0))
    return pl.pallas_call(
        _finalize_body,
        grid=(N // _RB,),
        in_specs=[
            row_spec,
            pl.BlockSpec((NC, _RB, DIM), lambda i: (0, i, 0)),
            pl.BlockSpec((_RB, NW), lambda i: (i, 0)),
            pl.BlockSpec((1, DIM), lambda i: (0, 0)),
            pl.BlockSpec((1, DIM), lambda i: (0, 0)),
        ],
        out_specs=row_spec,
        out_shape=jax.ShapeDtypeStruct((N, DIM), jnp.float32),
    )(h, part, degt, ln_g, ln_b)


def _head_body(h_ref, w1_ref, b1_ref, w2_ref, b2_ref, y_ref):
    t = jnp.dot(h_ref[...], w1_ref[...], preferred_element_type=jnp.float32)
    t = jnp.maximum(t + b1_ref[...], 0.0)
    y = jnp.dot(t, w2_ref[...], preferred_element_type=jnp.float32)
    y_ref[...] = y + b2_ref[...]


def _head(h, w1t, b1, w2t, b2):
    return pl.pallas_call(
        _head_body,
        grid=(N // _RB,),
        in_specs=[
            pl.BlockSpec((_RB, DIM), lambda i: (i, 0)),
            pl.BlockSpec((DIM, DIM // 2), lambda i: (0, 0)),
            pl.BlockSpec((1, DIM // 2), lambda i: (0, 0)),
            pl.BlockSpec((DIM // 2, 1), lambda i: (0, 0)),
            pl.BlockSpec((1, 1), lambda i: (0, 0)),
        ],
        out_specs=pl.BlockSpec((_RB, 1), lambda i: (i, 0)),
        out_shape=jax.ShapeDtypeStruct((N, 1), jnp.float32),
    )(h, w1t, b1, w2t, b2)


def _sc_mesh():
    return plsc.VectorSubcoreMesh(core_axis_name="c", subcore_axis_name="s")


def _deg_body(row_hbm, deg_hbm, hist, idxv):
    cid = lax.axis_index("c")
    sid = lax.axis_index("s")
    wid = sid * NC + cid
    zeros = jnp.zeros((L,), jnp.float32)
    ones = jnp.ones((L,), jnp.float32)

    def zero_body(i, _):
        hist[pl.ds(i * L, L)] = zeros
        return 0

    lax.fori_loop(0, N // L, zero_body, 0)

    def chunk_body(c, _):
        pltpu.sync_copy(row_hbm.at[pl.ds(wid * EPW + c * DK, DK)], idxv)

        def edge_body(i, _):
            iv = idxv[pl.ds(i * L, L)]
            plsc.addupdate_scatter(hist, [iv], ones)
            return 0

        lax.fori_loop(0, DK // L, edge_body, 0)
        return 0

    lax.fori_loop(0, NDCHUNK, chunk_body, 0)
    pltpu.sync_copy(hist, deg_hbm.at[wid])


def _degrees(row):
    fn = functools.partial(
        pl.kernel,
        out_type=jax.ShapeDtypeStruct((NW, N), jnp.float32),
        mesh=_sc_mesh(),
        scratch_types=[
            pltpu.VMEM((N,), jnp.float32),
            pltpu.VMEM((DK,), jnp.int32),
        ],
        compiler_params=pltpu.CompilerParams(needs_layout_passes=False),
    )(_deg_body)
    return fn(row)


def _edge_body(a_hbm, b_hbm, c_hbm, row_hbm, col_hbm, w2_hbm, b2_hbm,
               zero_hbm, out_hbm, row_v, col_v, av, bv, cv, w2v, b2v,
               acc, sem):
    cid = lax.axis_index("c")
    sid = lax.axis_index("s")
    wid = sid * NC + cid

    pltpu.sync_copy(zero_hbm.at[pl.ds(sid * RPT, RPT), :],
                    acc.at[pl.ds(sid * RPT, RPT), :])
    pltpu.sync_copy(w2_hbm, w2v)
    pltpu.sync_copy(b2_hbm, b2v)
    plsc.subcore_barrier()

    base = wid * EPW

    def chunk_body(gi, _):
        off = base + gi * EK
        pltpu.sync_copy(row_hbm.at[pl.ds(off, EK)], row_v)
        pltpu.sync_copy(col_hbm.at[pl.ds(off, EK)], col_v)
        ga = pltpu.async_copy(a_hbm.at[row_v], av, sem)
        gb = pltpu.async_copy(b_hbm.at[col_v], bv, sem)
        gc = pltpu.async_copy(c_hbm.at[col_v], cv, sem)
        ga.wait()
        gb.wait()
        gc.wait()
        b2l = b2v[...]

        def edge_body(e, _):
            acc16 = jnp.zeros((L,), jnp.float32)
            for j in range(DIM // L):
                sl = pl.ds(j * L, L)
                acc16 = acc16 + (jnp.maximum(av[e, sl] + bv[e, sl], 0.0)
                                 * w2v[sl])
            z = jnp.sum(acc16)
            gate = 1.0 / (1.0 + jnp.exp(-(jnp.full((L,), z) + b2l)))
            for j in range(DIM // L):
                sl = pl.ds(j * L, L)
                cv[e, sl] = cv[e, sl] * gate
            return 0

        lax.fori_loop(0, EK, edge_body, 0)
        pltpu.sync_copy(cv, acc.at[row_v], add=True)
        return 0

    lax.fori_loop(0, NCHUNK, chunk_body, 0)
    plsc.subcore_barrier()
    pltpu.sync_copy(acc.at[pl.ds(sid * RPT, RPT), :],
                    out_hbm.at[cid, pl.ds(sid * RPT, RPT), :])


def _edge_stage(a, b, c, row, col, w2, b2v, zeros):
    fn = functools.partial(
        pl.kernel,
        out_type=jax.ShapeDtypeStruct((NC, NP, DIM), jnp.float32),
        mesh=_sc_mesh(),
        scratch_types=[
            pltpu.VMEM((EK,), jnp.int32),
            pltpu.VMEM((EK,), jnp.int32),
            pltpu.VMEM((EK, DIM), jnp.float32),
            pltpu.VMEM((EK, DIM), jnp.float32),
            pltpu.VMEM((EK, DIM), jnp.float32),
            pltpu.VMEM((DIM,), jnp.float32),
            pltpu.VMEM((L,), jnp.float32),
            pltpu.VMEM_SHARED((NP, DIM), jnp.float32),
            pltpu.SemaphoreType.DMA,
        ],
        compiler_params=pltpu.CompilerParams(needs_layout_passes=False),
    )(_edge_body)
    return fn(a, b, c, row, col, w2, b2v, zeros)


def kernel(x, edge_index, params):
    p = params
    row = edge_index[0]
    col = edge_index[1]

    h = _encoder(
        x,
        p['enc_W1'].T, p['enc_b1'][None, :], p['bn1_g'][None, :],
        p['bn1_b'][None, :],
        p['enc_W2'].T, p['enc_b2'][None, :], p['bn2_g'][None, :],
        p['bn2_b'][None, :],
    )

    deg = _degrees(row)
    degt = deg.T
    zeros = jnp.zeros((NP, DIM), jnp.float32)

    for lp in p['layers']:
        wat = lp['g_W1'][:, :DIM].T
        wbt = lp['g_W1'][:, DIM:].T
        wt = lp['W'].T
        a, b, c = _tables(h, wat, wbt, lp['g_b1'][None, :], wt)
        b2v = jnp.full((L,), lp['g_b2'][0], jnp.float32)
        part = _edge_stage(a, b, c, row, col, lp['g_W2'][0], b2v, zeros)
        h = _finalize(h, part, degt, lp['ln_g'][None, :], lp['ln_b'][None, :])

    return _head(h, p['head_W1'].T, p['head_b1'][None, :],
                 p['head_W2'].T, p['head_b2'][None, :])


# speedup vs baseline: 3.8756x; 1.4090x over previous
import functools

import jax
import jax.numpy as jnp
from jax import lax
from jax.experimental import pallas as pl
from jax.experimental.pallas import tpu as pltpu
from jax.experimental.pallas import tpu_sc as plsc

N = 10000
DIM = 128
ENC = 256
E = 320000

NC = 2
NS = 16
L = 16
NW = NC * NS
EPW = E // NW
EK = 40
NCHUNK = EPW // EK
DK = 2000
NDCHUNK = EPW // DK
NP = 10240
RPT = NP // NS

_EPS = 1e-5


def _bn(x, g, b):
    mu = jnp.mean(x, axis=0, keepdims=True)
    var = jnp.mean((x - mu) * (x - mu), axis=0, keepdims=True)
    return (x - mu) * lax.rsqrt(var + _EPS) * g + b


def _encoder_body(x_ref, w1_ref, b1_ref, g1_ref, be1_ref, w2_ref, b2_ref,
                  g2_ref, be2_ref, h_ref):
    h1 = jnp.dot(x_ref[...], w1_ref[...], preferred_element_type=jnp.float32)
    h1 = h1 + b1_ref[...]
    h1 = jnp.maximum(_bn(h1, g1_ref[...], be1_ref[...]), 0.0)
    h2 = jnp.dot(h1, w2_ref[...], preferred_element_type=jnp.float32)
    h2 = h2 + b2_ref[...]
    h_ref[...] = jnp.maximum(_bn(h2, g2_ref[...], be2_ref[...]), 0.0)


def _encoder(x, w1t, b1, g1, be1, w2t, b2, g2, be2):
    return pl.pallas_call(
        _encoder_body,
        out_shape=jax.ShapeDtypeStruct((N, DIM), jnp.float32),
    )(x, w1t, b1, g1, be1, w2t, b2, g2, be2)


_RB = 2000


def _tables_body(h_ref, wa_ref, wb_ref, b1_ref, w_ref, a_ref, b_ref, c_ref):
    h = h_ref[...]
    a_ref[...] = jnp.dot(h, wa_ref[...], preferred_element_type=jnp.float32)
    b_ref[...] = jnp.dot(h, wb_ref[...],
                         preferred_element_type=jnp.float32) + b1_ref[...]
    c_ref[...] = jnp.dot(h, w_ref[...], preferred_element_type=jnp.float32)


def _tables(h, wat, wbt, b1, wt):
    row_spec = pl.BlockSpec((_RB, DIM), lambda i: (i, 0))
    full = pl.BlockSpec((DIM, DIM), lambda i: (0, 0))
    bias = pl.BlockSpec((1, DIM), lambda i: (0, 0))
    out = jax.ShapeDtypeStruct((N, DIM), jnp.float32)
    return pl.pallas_call(
        _tables_body,
        grid=(N // _RB,),
        in_specs=[row_spec, full, full, bias, full],
        out_specs=[row_spec, row_spec, row_spec],
        out_shape=[out, out, out],
    )(h, wat, wbt, b1, wt)


def _finalize_body(h_ref, p_ref, deg_ref, g_ref, b_ref, o_ref):
    d = jnp.sum(deg_ref[...], axis=1, keepdims=True)
    inv = 1.0 / jnp.maximum(d, 1.0)
    t = h_ref[...] + (p_ref[0] + p_ref[1]) * inv
    mu = jnp.mean(t, axis=1, keepdims=True)
    var = jnp.mean((t - mu) * (t - mu), axis=1, keepdims=True)
    ln = (t - mu) * lax.rsqrt(var + _EPS) * g_ref[...] + b_ref[...]
    o_ref[...] = jnp.maximum(ln, 0.0)


def _finalize(h, part, degt, ln_g, ln_b):
    row_spec = pl.BlockSpec((_RB, DIM), lambda i: (i, 0))
    return pl.pallas_call(
        _finalize_body,
        grid=(N // _RB,),
        in_specs=[
            row_spec,
            pl.BlockSpec((NC, _RB, DIM), lambda i: (0, i, 0)),
            pl.BlockSpec((_RB, NW), lambda i: (i, 0)),
            pl.BlockSpec((1, DIM), lambda i: (0, 0)),
            pl.BlockSpec((1, DIM), lambda i: (0, 0)),
        ],
        out_specs=row_spec,
        out_shape=jax.ShapeDtypeStruct((N, DIM), jnp.float32),
    )(h, part, degt, ln_g, ln_b)


def _head_body(h_ref, w1_ref, b1_ref, w2_ref, b2_ref, y_ref):
    t = jnp.dot(h_ref[...], w1_ref[...], preferred_element_type=jnp.float32)
    t = jnp.maximum(t + b1_ref[...], 0.0)
    y = jnp.dot(t, w2_ref[...], preferred_element_type=jnp.float32)
    y_ref[...] = y + b2_ref[...]


def _head(h, w1t, b1, w2t, b2):
    return pl.pallas_call(
        _head_body,
        grid=(N // _RB,),
        in_specs=[
            pl.BlockSpec((_RB, DIM), lambda i: (i, 0)),
            pl.BlockSpec((DIM, DIM // 2), lambda i: (0, 0)),
            pl.BlockSpec((1, DIM // 2), lambda i: (0, 0)),
            pl.BlockSpec((DIM // 2, 1), lambda i: (0, 0)),
            pl.BlockSpec((1, 1), lambda i: (0, 0)),
        ],
        out_specs=pl.BlockSpec((_RB, 1), lambda i: (i, 0)),
        out_shape=jax.ShapeDtypeStruct((N, 1), jnp.float32),
    )(h, w1t, b1, w2t, b2)


def _sc_mesh():
    return plsc.VectorSubcoreMesh(core_axis_name="c", subcore_axis_name="s")


def _deg_body(row_hbm, deg_hbm, hist, idxv):
    cid = lax.axis_index("c")
    sid = lax.axis_index("s")
    wid = sid * NC + cid
    zeros = jnp.zeros((L,), jnp.float32)
    ones = jnp.ones((L,), jnp.float32)

    def zero_body(i, _):
        hist[pl.ds(i * L, L)] = zeros
        return 0

    lax.fori_loop(0, N // L, zero_body, 0)

    def chunk_body(c, _):
        pltpu.sync_copy(row_hbm.at[pl.ds(wid * EPW + c * DK, DK)], idxv)

        def edge_body(i, _):
            iv = idxv[pl.ds(i * L, L)]
            plsc.addupdate_scatter(hist, [iv], ones)
            return 0

        lax.fori_loop(0, DK // L, edge_body, 0)
        return 0

    lax.fori_loop(0, NDCHUNK, chunk_body, 0)
    pltpu.sync_copy(hist, deg_hbm.at[wid])


def _degrees(row):
    fn = functools.partial(
        pl.kernel,
        out_type=jax.ShapeDtypeStruct((NW, N), jnp.float32),
        mesh=_sc_mesh(),
        scratch_types=[
            pltpu.VMEM((N,), jnp.float32),
            pltpu.VMEM((DK,), jnp.int32),
        ],
        compiler_params=pltpu.CompilerParams(needs_layout_passes=False),
    )(_deg_body)
    return fn(row)


def _edge_body(a_hbm, b_hbm, c_hbm, idx_hbm, w2_hbm, b2_hbm,
               zero_hbm, out_hbm,
               rcv0, rcv1, av0, av1, bv0, bv1, cv0, cv1, sv0, sv1,
               sidx0, sidx1, w2v, b2v, acc,
               gsem0, gsem1, ssem0, ssem1):
    cid = lax.axis_index("c")
    sid = lax.axis_index("s")
    wid = sid * NC + cid
    rcv = (rcv0, rcv1)
    av = (av0, av1)
    bv = (bv0, bv1)
    cv = (cv0, cv1)
    sv = (sv0, sv1)
    sidx = (sidx0, sidx1)
    gsem = (gsem0, gsem1)
    ssem = (ssem0, ssem1)

    pltpu.sync_copy(zero_hbm.at[pl.ds(sid * RPT, RPT), :],
                    acc.at[pl.ds(sid * RPT, RPT), :])
    pltpu.sync_copy(w2_hbm, w2v)
    pltpu.sync_copy(b2_hbm, b2v)
    plsc.subcore_barrier()

    cbase = wid * NCHUNK

    def load_idx(g, b):
        pltpu.sync_copy(idx_hbm.at[cbase + g], rcv[b])

    def fire_gathers(b):
        pltpu.async_copy(a_hbm.at[rcv[b].at[0]], av[b], gsem[b])
        pltpu.async_copy(b_hbm.at[rcv[b].at[1]], bv[b], gsem[b])
        pltpu.async_copy(c_hbm.at[rcv[b].at[1]], cv[b], gsem[b])

    def wait_gathers(b):
        pltpu.make_async_copy(a_hbm.at[pl.ds(0, EK), :], av[b], gsem[b]).wait()
        pltpu.make_async_copy(a_hbm.at[pl.ds(0, EK), :], bv[b], gsem[b]).wait()
        pltpu.make_async_copy(a_hbm.at[pl.ds(0, EK), :], cv[b], gsem[b]).wait()

    def compute(b):
        b2l = b2v[...]
        avb, bvb, cvb, svb = av[b], bv[b], cv[b], sv[b]

        def edge_body(e, _):
            acc16 = jnp.zeros((L,), jnp.float32)
            for j in range(DIM // L):
                sl = pl.ds(j * L, L)
                acc16 = acc16 + (jnp.maximum(avb[e, sl] + bvb[e, sl], 0.0)
                                 * w2v[sl])
            z = jnp.sum(acc16)
            gate = 1.0 / (1.0 + jnp.exp(-(jnp.full((L,), z) + b2l)))
            for j in range(DIM // L):
                sl = pl.ds(j * L, L)
                svb[e, sl] = cvb[e, sl] * gate
            return 0

        lax.fori_loop(0, EK, edge_body, 0)

    def fire_scatter(b):
        for st in (0, L, EK - L):
            sidx[b][pl.ds(st, L)] = rcv[b][0, pl.ds(st, L)]
        pltpu.async_copy(sv[b], acc.at[sidx[b]], ssem[b], add=True)

    def wait_scatter(b):
        pltpu.make_async_copy(a_hbm.at[pl.ds(0, EK), :], sv[b], ssem[b]).wait()

    load_idx(0, 0)
    fire_gathers(0)
    load_idx(1, 1)
    fire_gathers(1)
    wait_gathers(0)
    compute(0)
    fire_scatter(0)
    load_idx(2, 0)
    fire_gathers(0)
    wait_gathers(1)
    compute(1)
    fire_scatter(1)

    def pair_body(i2, _):
        for b in (0, 1):
            g = 2 * i2 + b
            load_idx(g + 1, 1 - b)
            fire_gathers(1 - b)
            wait_gathers(b)
            wait_scatter(b)
            compute(b)
            fire_scatter(b)
        return 0

    lax.fori_loop(1, NCHUNK // 2 - 1, pair_body, 0)

    load_idx(NCHUNK - 1, 1)
    fire_gathers(1)
    wait_gathers(0)
    wait_scatter(0)
    compute(0)
    fire_scatter(0)
    wait_gathers(1)
    wait_scatter(1)
    compute(1)
    fire_scatter(1)
    wait_scatter(0)
    wait_scatter(1)

    plsc.subcore_barrier()
    pltpu.sync_copy(acc.at[pl.ds(sid * RPT, RPT), :],
                    out_hbm.at[cid, pl.ds(sid * RPT, RPT), :])


def _edge_stage(a, b, c, idx3, w2, b2v, zeros):
    fn = functools.partial(
        pl.kernel,
        out_type=jax.ShapeDtypeStruct((NC, NP, DIM), jnp.float32),
        mesh=_sc_mesh(),
        scratch_types=[
            pltpu.VMEM((2, EK), jnp.int32),
            pltpu.VMEM((2, EK), jnp.int32),
            pltpu.VMEM((EK, DIM), jnp.float32),
            pltpu.VMEM((EK, DIM), jnp.float32),
            pltpu.VMEM((EK, DIM), jnp.float32),
            pltpu.VMEM((EK, DIM), jnp.float32),
            pltpu.VMEM((EK, DIM), jnp.float32),
            pltpu.VMEM((EK, DIM), jnp.float32),
            pltpu.VMEM((EK, DIM), jnp.float32),
            pltpu.VMEM((EK, DIM), jnp.float32),
            pltpu.VMEM((EK,), jnp.int32),
            pltpu.VMEM((EK,), jnp.int32),
            pltpu.VMEM((DIM,), jnp.float32),
            pltpu.VMEM((L,), jnp.float32),
            pltpu.VMEM_SHARED((NP, DIM), jnp.float32),
            pltpu.SemaphoreType.DMA,
            pltpu.SemaphoreType.DMA,
            pltpu.SemaphoreType.DMA,
            pltpu.SemaphoreType.DMA,
        ],
        compiler_params=pltpu.CompilerParams(needs_layout_passes=False),
    )(_edge_body)
    return fn(a, b, c, idx3, w2, b2v, zeros)


def kernel(x, edge_index, params):
    p = params
    row = edge_index[0]
    col = edge_index[1]

    h = _encoder(
        x,
        p['enc_W1'].T, p['enc_b1'][None, :], p['bn1_g'][None, :],
        p['bn1_b'][None, :],
        p['enc_W2'].T, p['enc_b2'][None, :], p['bn2_g'][None, :],
        p['bn2_b'][None, :],
    )

    deg = _degrees(row)
    degt = deg.T
    zeros = jnp.zeros((NP, DIM), jnp.float32)
    idx3 = jnp.stack([row.reshape(NW, NCHUNK, EK),
                      col.reshape(NW, NCHUNK, EK)],
                     axis=2).reshape(NW * NCHUNK, 2, EK)

    for lp in p['layers']:
        wat = lp['g_W1'][:, :DIM].T
        wbt = lp['g_W1'][:, DIM:].T
        wt = lp['W'].T
        a, b, c = _tables(h, wat, wbt, lp['g_b1'][None, :], wt)
        b2v = jnp.full((L,), lp['g_b2'][0], jnp.float32)
        part = _edge_stage(a, b, c, idx3, lp['g_W2'][0], b2v, zeros)
        h = _finalize(h, part, degt, lp['ln_g'][None, :], lp['ln_b'][None, :])

    return _head(h, p['head_W1'].T, p['head_b1'][None, :],
                 p['head_W2'].T, p['head_b2'][None, :])


# speedup vs baseline: 6.0919x; 1.5719x over previous
import functools

import jax
import jax.numpy as jnp
from jax import lax
from jax.experimental import pallas as pl
from jax.experimental.pallas import tpu as pltpu
from jax.experimental.pallas import tpu_sc as plsc

N = 10000
DIM = 128
ENC = 256
E = 320000

NC = 2
NS = 16
L = 16
NW = NC * NS
EPW = E // NW
EK = 40
NCHUNK = EPW // EK
DK = 2000
NDCHUNK = EPW // DK
NP = 10240
RPT = NP // NS

_EPS = 1e-5


def _bn(x, g, b):
    mu = jnp.mean(x, axis=0, keepdims=True)
    var = jnp.mean((x - mu) * (x - mu), axis=0, keepdims=True)
    return (x - mu) * lax.rsqrt(var + _EPS) * g + b


def _encoder_body(x_ref, w1_ref, b1_ref, g1_ref, be1_ref, w2_ref, b2_ref,
                  g2_ref, be2_ref, h_ref):
    h1 = jnp.dot(x_ref[...], w1_ref[...], preferred_element_type=jnp.float32)
    h1 = h1 + b1_ref[...]
    h1 = jnp.maximum(_bn(h1, g1_ref[...], be1_ref[...]), 0.0)
    h2 = jnp.dot(h1, w2_ref[...], preferred_element_type=jnp.float32)
    h2 = h2 + b2_ref[...]
    h_ref[...] = jnp.maximum(_bn(h2, g2_ref[...], be2_ref[...]), 0.0)


def _encoder(x, w1t, b1, g1, be1, w2t, b2, g2, be2):
    return pl.pallas_call(
        _encoder_body,
        out_shape=jax.ShapeDtypeStruct((N, DIM), jnp.float32),
    )(x, w1t, b1, g1, be1, w2t, b2, g2, be2)


_RB = 2000


def _tables_body(h_ref, wa_ref, wb_ref, b1_ref, w_ref, a_ref, b_ref, c_ref):
    h = h_ref[...]
    a_ref[...] = jnp.dot(h, wa_ref[...], preferred_element_type=jnp.float32)
    b_ref[...] = jnp.dot(h, wb_ref[...],
                         preferred_element_type=jnp.float32) + b1_ref[...]
    c_ref[...] = jnp.dot(h, w_ref[...], preferred_element_type=jnp.float32)


def _tables(h, wat, wbt, b1, wt):
    row_spec = pl.BlockSpec((_RB, DIM), lambda i: (i, 0))
    full = pl.BlockSpec((DIM, DIM), lambda i: (0, 0))
    bias = pl.BlockSpec((1, DIM), lambda i: (0, 0))
    out = jax.ShapeDtypeStruct((N, DIM), jnp.float32)
    return pl.pallas_call(
        _tables_body,
        grid=(N // _RB,),
        in_specs=[row_spec, full, full, bias, full],
        out_specs=[row_spec, row_spec, row_spec],
        out_shape=[out, out, out],
    )(h, wat, wbt, b1, wt)


def _finalize_body(h_ref, p_ref, deg_ref, g_ref, b_ref, o_ref):
    d = jnp.sum(deg_ref[...], axis=1, keepdims=True)
    inv = 1.0 / jnp.maximum(d, 1.0)
    t = h_ref[...] + (p_ref[0] + p_ref[1]) * inv
    mu = jnp.mean(t, axis=1, keepdims=True)
    var = jnp.mean((t - mu) * (t - mu), axis=1, keepdims=True)
    ln = (t - mu) * lax.rsqrt(var + _EPS) * g_ref[...] + b_ref[...]
    o_ref[...] = jnp.maximum(ln, 0.0)


def _finalize(h, part, degt, ln_g, ln_b):
    row_spec = pl.BlockSpec((_RB, DIM), lambda i: (i, 0))
    return pl.pallas_call(
        _finalize_body,
        grid=(N // _RB,),
        in_specs=[
            row_spec,
            pl.BlockSpec((NC, _RB, DIM), lambda i: (0, i, 0)),
            pl.BlockSpec((_RB, NW), lambda i: (i, 0)),
            pl.BlockSpec((1, DIM), lambda i: (0, 0)),
            pl.BlockSpec((1, DIM), lambda i: (0, 0)),
        ],
        out_specs=row_spec,
        out_shape=jax.ShapeDtypeStruct((N, DIM), jnp.float32),
    )(h, part, degt, ln_g, ln_b)


def _head_body(h_ref, w1_ref, b1_ref, w2_ref, b2_ref, y_ref):
    t = jnp.dot(h_ref[...], w1_ref[...], preferred_element_type=jnp.float32)
    t = jnp.maximum(t + b1_ref[...], 0.0)
    y = jnp.dot(t, w2_ref[...], preferred_element_type=jnp.float32)
    y_ref[...] = y + b2_ref[...]


def _head(h, w1t, b1, w2t, b2):
    return pl.pallas_call(
        _head_body,
        grid=(N // _RB,),
        in_specs=[
            pl.BlockSpec((_RB, DIM), lambda i: (i, 0)),
            pl.BlockSpec((DIM, DIM // 2), lambda i: (0, 0)),
            pl.BlockSpec((1, DIM // 2), lambda i: (0, 0)),
            pl.BlockSpec((DIM // 2, 1), lambda i: (0, 0)),
            pl.BlockSpec((1, 1), lambda i: (0, 0)),
        ],
        out_specs=pl.BlockSpec((_RB, 1), lambda i: (i, 0)),
        out_shape=jax.ShapeDtypeStruct((N, 1), jnp.float32),
    )(h, w1t, b1, w2t, b2)


def _sc_mesh():
    return plsc.VectorSubcoreMesh(core_axis_name="c", subcore_axis_name="s")


def _deg_body(row_hbm, deg_hbm, hist, idxv):
    cid = lax.axis_index("c")
    sid = lax.axis_index("s")
    wid = sid * NC + cid
    zeros = jnp.zeros((L,), jnp.float32)
    ones = jnp.ones((L,), jnp.float32)

    def zero_body(i, _):
        hist[pl.ds(i * L, L)] = zeros
        return 0

    lax.fori_loop(0, N // L, zero_body, 0)

    def chunk_body(c, _):
        pltpu.sync_copy(row_hbm.at[pl.ds(wid * EPW + c * DK, DK)], idxv)

        def edge_body(i, _):
            iv = idxv[pl.ds(i * L, L)]
            plsc.addupdate_scatter(hist, [iv], ones)
            return 0

        lax.fori_loop(0, DK // L, edge_body, 0)
        return 0

    lax.fori_loop(0, NDCHUNK, chunk_body, 0)
    pltpu.sync_copy(hist, deg_hbm.at[wid])


def _degrees(row):
    fn = functools.partial(
        pl.kernel,
        out_type=jax.ShapeDtypeStruct((NW, N), jnp.float32),
        mesh=_sc_mesh(),
        scratch_types=[
            pltpu.VMEM((N,), jnp.float32),
            pltpu.VMEM((DK,), jnp.int32),
        ],
        compiler_params=pltpu.CompilerParams(needs_layout_passes=False),
    )(_deg_body)
    return fn(row)


def _edge_body(a_hbm, b_hbm, c_hbm, idx_hbm, w2_hbm, b2_hbm,
               zero_hbm, out_hbm,
               rcv0, rcv1, av0, av1, bv0, bv1, cv0, cv1, sv0, sv1,
               sidx0, sidx1, w2v, b2v, acc,
               gsem0, gsem1, ssem0, ssem1):
    cid = lax.axis_index("c")
    sid = lax.axis_index("s")
    wid = sid * NC + cid
    rcv = (rcv0, rcv1)
    av = (av0, av1)
    bv = (bv0, bv1)
    cv = (cv0, cv1)
    sv = (sv0, sv1)
    sidx = (sidx0, sidx1)
    gsem = (gsem0, gsem1)
    ssem = (ssem0, ssem1)

    pltpu.sync_copy(zero_hbm.at[pl.ds(sid * RPT, RPT), :],
                    acc.at[pl.ds(sid * RPT, RPT), :])
    pltpu.sync_copy(w2_hbm, w2v)
    pltpu.sync_copy(b2_hbm, b2v)
    plsc.subcore_barrier()

    cbase = wid * NCHUNK

    def load_idx(g, b):
        pltpu.sync_copy(idx_hbm.at[cbase + g], rcv[b])

    def fire_gathers(b):
        pltpu.async_copy(a_hbm.at[rcv[b].at[0]], av[b], gsem[b])
        pltpu.async_copy(b_hbm.at[rcv[b].at[1]], bv[b], gsem[b])
        pltpu.async_copy(c_hbm.at[rcv[b].at[1]], cv[b], gsem[b])

    def wait_gathers(b):
        pltpu.make_async_copy(a_hbm.at[pl.ds(0, EK), :], av[b], gsem[b]).wait()
        pltpu.make_async_copy(a_hbm.at[pl.ds(0, EK), :], bv[b], gsem[b]).wait()
        pltpu.make_async_copy(a_hbm.at[pl.ds(0, EK), :], cv[b], gsem[b]).wait()

    def compute(b):
        b2l = b2v[...]
        avb, bvb, cvb, svb = av[b], bv[b], cv[b], sv[b]

        @plsc.parallel_loop(0, EK, step=1, unroll=4)
        def edge_body(e):
            acc16 = jnp.zeros((L,), jnp.float32)
            for j in range(DIM // L):
                sl = pl.ds(j * L, L)
                acc16 = acc16 + (jnp.maximum(avb[e, sl] + bvb[e, sl], 0.0)
                                 * w2v[sl])
            z = jnp.sum(acc16)
            gate = 1.0 / (1.0 + jnp.exp(-(jnp.full((L,), z) + b2l)))
            for j in range(DIM // L):
                sl = pl.ds(j * L, L)
                svb[e, sl] = cvb[e, sl] * gate

    def fire_scatter(b):
        for st in (0, L, EK - L):
            sidx[b][pl.ds(st, L)] = rcv[b][0, pl.ds(st, L)]
        pltpu.async_copy(sv[b], acc.at[sidx[b]], ssem[b], add=True)

    def wait_scatter(b):
        pltpu.make_async_copy(a_hbm.at[pl.ds(0, EK), :], sv[b], ssem[b]).wait()

    load_idx(0, 0)
    fire_gathers(0)
    load_idx(1, 1)
    fire_gathers(1)
    wait_gathers(0)
    compute(0)
    fire_scatter(0)
    load_idx(2, 0)
    fire_gathers(0)
    wait_gathers(1)
    compute(1)
    fire_scatter(1)

    def pair_body(i2, _):
        for b in (0, 1):
            g = 2 * i2 + b
            load_idx(g + 1, 1 - b)
            fire_gathers(1 - b)
            wait_gathers(b)
            wait_scatter(b)
            compute(b)
            fire_scatter(b)
        return 0

    lax.fori_loop(1, NCHUNK // 2 - 1, pair_body, 0)

    load_idx(NCHUNK - 1, 1)
    fire_gathers(1)
    wait_gathers(0)
    wait_scatter(0)
    compute(0)
    fire_scatter(0)
    wait_gathers(1)
    wait_scatter(1)
    compute(1)
    fire_scatter(1)
    wait_scatter(0)
    wait_scatter(1)

    plsc.subcore_barrier()
    pltpu.sync_copy(acc.at[pl.ds(sid * RPT, RPT), :],
                    out_hbm.at[cid, pl.ds(sid * RPT, RPT), :])


def _edge_stage(a, b, c, idx3, w2, b2v, zeros):
    fn = functools.partial(
        pl.kernel,
        out_type=jax.ShapeDtypeStruct((NC, NP, DIM), jnp.float32),
        mesh=_sc_mesh(),
        scratch_types=[
            pltpu.VMEM((2, EK), jnp.int32),
            pltpu.VMEM((2, EK), jnp.int32),
            pltpu.VMEM((EK, DIM), jnp.float32),
            pltpu.VMEM((EK, DIM), jnp.float32),
            pltpu.VMEM((EK, DIM), jnp.float32),
            pltpu.VMEM((EK, DIM), jnp.float32),
            pltpu.VMEM((EK, DIM), jnp.float32),
            pltpu.VMEM((EK, DIM), jnp.float32),
            pltpu.VMEM((EK, DIM), jnp.float32),
            pltpu.VMEM((EK, DIM), jnp.float32),
            pltpu.VMEM((EK,), jnp.int32),
            pltpu.VMEM((EK,), jnp.int32),
            pltpu.VMEM((DIM,), jnp.float32),
            pltpu.VMEM((L,), jnp.float32),
            pltpu.VMEM_SHARED((NP, DIM), jnp.float32),
            pltpu.SemaphoreType.DMA,
            pltpu.SemaphoreType.DMA,
            pltpu.SemaphoreType.DMA,
            pltpu.SemaphoreType.DMA,
        ],
        compiler_params=pltpu.CompilerParams(needs_layout_passes=False),
    )(_edge_body)
    return fn(a, b, c, idx3, w2, b2v, zeros)


def kernel(x, edge_index, params):
    p = params
    row = edge_index[0]
    col = edge_index[1]

    h = _encoder(
        x,
        p['enc_W1'].T, p['enc_b1'][None, :], p['bn1_g'][None, :],
        p['bn1_b'][None, :],
        p['enc_W2'].T, p['enc_b2'][None, :], p['bn2_g'][None, :],
        p['bn2_b'][None, :],
    )

    deg = _degrees(row)
    degt = deg.T
    zeros = jnp.zeros((NP, DIM), jnp.float32)
    idx3 = jnp.stack([row.reshape(NW, NCHUNK, EK),
                      col.reshape(NW, NCHUNK, EK)],
                     axis=2).reshape(NW * NCHUNK, 2, EK)

    for lp in p['layers']:
        wat = lp['g_W1'][:, :DIM].T
        wbt = lp['g_W1'][:, DIM:].T
        wt = lp['W'].T
        a, b, c = _tables(h, wat, wbt, lp['g_b1'][None, :], wt)
        b2v = jnp.full((L,), lp['g_b2'][0], jnp.float32)
        part = _edge_stage(a, b, c, idx3, lp['g_W2'][0], b2v, zeros)
        h = _finalize(h, part, degt, lp['ln_g'][None, :], lp['ln_b'][None, :])

    return _head(h, p['head_W1'].T, p['head_b1'][None, :],
                 p['head_W2'].T, p['head_b2'][None, :])
